# baseline (speedup 1.0000x reference)
import functools

import jax
import jax.numpy as jnp
from jax import lax
from jax.experimental import pallas as pl
from jax.experimental.pallas import tpu as pltpu
from jax.experimental.pallas import tpu_sc as plsc

NPAD = 10240
TILES = 32
EPT = 10240
EPAD = TILES * EPT
K = 512
NB = EPT // K
NG = K // 16
NSB = NB // 2
NR = NPAD // 16

_CP = pltpu.CompilerParams(needs_layout_passes=False, use_tc_tiling_on_sc=False)


def _proj_nodes_kernel(x_ref, wi_ref, wj_ref, a_ref, b_ref):
    xv = x_ref[...]
    a_ref[...] = (2.0 * jnp.dot(xv, wi_ref[...],
                                preferred_element_type=jnp.float32)).astype(jnp.bfloat16)
    b_ref[...] = (2.0 * jnp.dot(xv, wj_ref[...],
                                preferred_element_type=jnp.float32)).astype(jnp.bfloat16)


def _proj_edges_kernel(ea_ref, w2_ref, c_ref):
    c_ref[...] = (2.0 * jnp.dot(ea_ref[...], w2_ref[...],
                                preferred_element_type=jnp.float32)).astype(jnp.bfloat16)


def _f32(x):
    return plsc.bitcast(x, jnp.float32)


def _make_sc_kernels(H):
    mesh = plsc.VectorSubcoreMesh(core_axis_name="c", subcore_axis_name="s")

    @functools.partial(
        pl.kernel,
        out_type=(
            jax.ShapeDtypeStruct((EPAD,), jnp.float32),
            jax.ShapeDtypeStruct((2, NR, 16), jnp.float32),
        ),
        mesh=mesh,
        scratch_types=(
            pltpu.VMEM((K,), jnp.int32),
            pltpu.VMEM((K,), jnp.int32),
            pltpu.VMEM((K,), jnp.int32),
            pltpu.VMEM((K,), jnp.int32),
            pltpu.VMEM((K,), jnp.int32),
            pltpu.VMEM((K, H), jnp.bfloat16),
            pltpu.VMEM((K, H), jnp.bfloat16),
            pltpu.VMEM((K, H), jnp.bfloat16),
            pltpu.VMEM((K, H), jnp.bfloat16),
            pltpu.VMEM((K // 2, 2 * H), jnp.bfloat16),
            pltpu.VMEM((K // 2, 2 * H), jnp.bfloat16),
            pltpu.VMEM((16 * K,), jnp.float32),
            pltpu.VMEM((K,), jnp.float32),
            pltpu.VMEM((K,), jnp.float32),
            pltpu.VMEM((NR, 16), jnp.float32),
            pltpu.VMEM((5, 128), jnp.int32),
            pltpu.VMEM((H,), jnp.float32),
            pltpu.VMEM_SHARED((NR, 16), jnp.float32),
            pltpu.SemaphoreType.DMA,
            pltpu.SemaphoreType.DMA,
            pltpu.SemaphoreType.DMA,
            pltpu.SemaphoreType.DMA,
            pltpu.SemaphoreType.DMA,
            pltpu.SemaphoreType.DMA,
        ),
        compiler_params=_CP,
    )
    def edge_kernel(a_hbm, b_hbm, c_hbm, i_hbm, j_hbm, ws_hbm,
                    ex_hbm, dn_hbm,
                    idxi0, idxi1, idxj0, idxj1, cidx,
                    bufa0, bufa1, bufb0, bufb1, bufc0, bufc1,
                    wbuf, exv0, exv1, dnv, rowidx, ws_v, shared_dn,
                    semi0, semi1, semg0, semg1, semo0, semo1):
        cid = lax.axis_index("c")
        sid = lax.axis_index("s")
        wid = cid * 16 + sid
        e0 = wid * EPT
        idxi = (idxi0, idxi1)
        idxj = (idxj0, idxj1)
        bufa = (bufa0, bufa1)
        bufb = (bufb0, bufb1)
        bufc = (bufc0, bufc1)
        exv = (exv0, exv1)
        semi = (semi0, semi1)
        semg = (semg0, semg1)
        semo = (semo0, semo1)

        def idx_descs(par, bb):
            eb = e0 + bb * K
            return (
                pltpu.make_async_copy(i_hbm.at[pl.ds(eb, K)], idxi[par], semi[par]),
                pltpu.make_async_copy(j_hbm.at[pl.ds(eb, K)], idxj[par], semi[par]),
            )

        def gather_descs(par, bb):
            eb = e0 + bb * K
            return (
                pltpu.make_async_copy(a_hbm.at[idxi[par]], bufa[par], semg[par]),
                pltpu.make_async_copy(b_hbm.at[idxj[par]], bufb[par], semg[par]),
                pltpu.make_async_copy(c_hbm.at[pl.ds(eb // 2, K // 2)],
                                      bufc[par], semg[par]),
            )

        def out_desc(par, bb):
            eb = e0 + bb * K
            return pltpu.make_async_copy(exv[par], ex_hbm.at[pl.ds(eb, K)], semo[par])

        pltpu.sync_copy(ws_hbm, ws_v)
        wsv = [ws_v[pl.ds(16 * k, 16)] for k in range(4)]
        ws2v = [2.0 * w for w in wsv]
        iota = lax.iota(jnp.int32, 16)
        iota_k = iota * K
        mask_hi = jnp.full((16,), -65536, jnp.int32)
        zero16 = jnp.zeros((16,), jnp.float32)

        @plsc.parallel_loop(0, NR)
        def _(r):
            dnv[r] = zero16

        for p in range(5):
            for o in range(8):
                rowidx[p, pl.ds(o * 16, 16)] = iota + (p * 128 + o * 16)

        @pl.when(sid == 0)
        def _():
            pltpu.sync_copy(dnv, shared_dn)
        plsc.subcore_barrier()

        def compute(par, bb):
            for o in range(NG):
                sl = pl.ds(o * 16, 16)
                cidx[sl] = idxi[par][sl]

            ba, bb_, bc = bufa[par], bufb[par], bufc[par]

            @plsc.parallel_loop(0, K // 2, unroll=2)
            def _(rr):
                e2 = rr * 2
                for sub in range(2):
                    e = e2 + sub
                    acc = None
                    for ch in range(2):
                        sl = pl.ds(32 * ch, 32)
                        va = plsc.bitcast(ba[e, sl], jnp.int32)
                        vb = plsc.bitcast(bb_[e, sl], jnp.int32)
                        vc = plsc.bitcast(
                            bc[rr, pl.ds(sub * 64 + 32 * ch, 32)], jnp.int32)
                        for half in range(2):
                            if half == 0:
                                ae = _f32(va << 16)
                                be = _f32(vb << 16)
                                ce = _f32(vc << 16)
                            else:
                                ae = _f32(va & mask_hi)
                                be = _f32(vb & mask_hi)
                                ce = _f32(vc & mask_hi)
                            k = ch * 2 + half
                            s = (ae + be) + ce
                            t = jnp.exp(s)
                            r = wsv[k] - ws2v[k] / (t + 1.0)
                            acc = r if acc is None else acc + r
                    plsc.store_scatter(wbuf, [iota_k + e], acc)

            ev = exv[par]

            @plsc.parallel_loop(0, NG, unroll=2)
            def _(g):
                gb = g * 16
                accs = [wbuf[pl.ds(l * K + gb, 16)] for l in range(4)]
                for q in range(1, 4):
                    for l in range(4):
                        accs[l] = accs[l] + wbuf[pl.ds((q * 4 + l) * K + gb, 16)]
                acc = (accs[0] + accs[1]) + (accs[2] + accs[3])
                ex16 = jnp.exp(acc)
                ev[pl.ds(gb, 16)] = ex16
                dst = cidx[pl.ds(gb, 16)]
                plsc.addupdate_scatter(dnv, [dst >> 4, dst & 15], ex16)

        for d in idx_descs(0, 0):
            d.start()
        for d in idx_descs(0, 0):
            d.wait()
        for d in gather_descs(0, 0):
            d.start()
        for d in idx_descs(1, 1):
            d.start()

        def hyper_body(hb, carry):
            for u in (0, 1):
                bb = hb * 2 + u
                par = u
                opar = 1 - u

                if u == 0:
                    for d in idx_descs(opar, bb + 1):
                        d.wait()
                    for d in gather_descs(opar, bb + 1):
                        d.start()
                else:
                    @pl.when(hb < NSB - 1)
                    def _():
                        for d in idx_descs(opar, bb + 1):
                            d.wait()
                        for d in gather_descs(opar, bb + 1):
                            d.start()

                for d in gather_descs(par, bb):
                    d.wait()

                @pl.when(hb >= 1)
                def _():
                    out_desc(par, bb - 2).wait()

                compute(par, bb)

                @pl.when(hb < NSB - 1)
                def _():
                    for d in idx_descs(par, bb + 2):
                        d.start()

                out_desc(par, bb).start()
            return carry

        lax.fori_loop(0, NSB, hyper_body, 0)
        out_desc(0, NB - 2).wait()
        out_desc(1, NB - 1).wait()

        for p in range(5):
            pltpu.sync_copy(dnv.at[pl.ds(p * 128, 128)],
                            shared_dn.at[rowidx.at[p]], add=True)
        plsc.subcore_barrier()
        pltpu.sync_copy(shared_dn.at[pl.ds(sid * (NR // 16), NR // 16)],
                        dn_hbm.at[cid, pl.ds(sid * (NR // 16), NR // 16)])

    @functools.partial(
        pl.kernel,
        out_type=jax.ShapeDtypeStruct((EPAD,), jnp.float32),
        mesh=mesh,
        scratch_types=(
            pltpu.VMEM((NR, 16), jnp.float32),
            pltpu.VMEM((NR, 16), jnp.float32),
            pltpu.VMEM((EPT,), jnp.float32),
            pltpu.VMEM((EPT,), jnp.int32),
            pltpu.VMEM((EPT,), jnp.float32),
            pltpu.SemaphoreType.DMA,
        ),
        compiler_params=_CP,
    )
    def norm_kernel(dn_hbm, ex_hbm, i_hbm, al_hbm,
                    dn0, dn1, ex_v, idx_v, al_v, sem):
        cid = lax.axis_index("c")
        sid = lax.axis_index("s")
        wid = cid * 16 + sid
        e0 = wid * EPT
        cp1 = pltpu.make_async_copy(dn_hbm.at[0], dn0, sem)
        cp2 = pltpu.make_async_copy(dn_hbm.at[1], dn1, sem)
        cp3 = pltpu.make_async_copy(ex_hbm.at[pl.ds(e0, EPT)], ex_v, sem)
        cp4 = pltpu.make_async_copy(i_hbm.at[pl.ds(e0, EPT)], idx_v, sem)
        for cp in (cp1, cp2, cp3, cp4):
            cp.start()
        cp1.wait()
        cp2.wait()

        @plsc.parallel_loop(0, NR)
        def _(r):
            dn0[r] = 1.0 / ((dn0[r] + dn1[r]) + 1e-16)

        cp3.wait()
        cp4.wait()

        @plsc.parallel_loop(0, EPT // 16, unroll=2)
        def _(g):
            sl = pl.ds(g * 16, 16)
            dst = idx_v[sl]
            inv = plsc.load_gather(dn0, [dst >> 4, dst & 15])
            al_v[sl] = ex_v[sl] * inv

        pltpu.sync_copy(al_v, al_hbm.at[pl.ds(e0, EPT)])

    return edge_kernel, norm_kernel


def kernel(x, edge_index, edge_attr, W_i, W_j, W_e, w_s):
    N, C = x.shape
    E, DE = edge_attr.shape
    H = W_i.shape[0]

    i_pad = jnp.concatenate(
        [edge_index[1], jnp.full((EPAD - E,), N, jnp.int32)], axis=0)
    j_pad = jnp.concatenate(
        [edge_index[0], jnp.zeros((EPAD - E,), jnp.int32)], axis=0)

    ws = w_s.reshape(H)
    ws_re = jnp.concatenate(
        [ws[0:32:2], ws[1:32:2], ws[32:64:2], ws[33:64:2]])

    nblk = 1024
    a2, b2 = pl.pallas_call(
        _proj_nodes_kernel,
        grid=(NPAD // nblk,),
        in_specs=[
            pl.BlockSpec((nblk, C), lambda g: (g, 0)),
            pl.BlockSpec((C, H), lambda g: (0, 0)),
            pl.BlockSpec((C, H), lambda g: (0, 0)),
        ],
        out_specs=[
            pl.BlockSpec((nblk, H), lambda g: (g, 0)),
            pl.BlockSpec((nblk, H), lambda g: (g, 0)),
        ],
        out_shape=[
            jax.ShapeDtypeStruct((NPAD, H), jnp.bfloat16),
            jax.ShapeDtypeStruct((NPAD, H), jnp.bfloat16),
        ],
    )(x, W_i.T, W_j.T)

    wet = W_e.T
    zde = jnp.zeros((DE, H), jnp.float32)
    w2 = jnp.concatenate([
        jnp.concatenate([wet, zde], axis=1),
        jnp.concatenate([zde, wet], axis=1),
    ], axis=0)

    ea_v = edge_attr.reshape(E // 2, 2 * DE)
    eblk = 16384
    c2 = pl.pallas_call(
        _proj_edges_kernel,
        grid=(EPAD // eblk,),
        in_specs=[
            pl.BlockSpec((eblk // 2, 2 * DE), lambda g: (g, 0)),
            pl.BlockSpec((2 * DE, 2 * H), lambda g: (0, 0)),
        ],
        out_specs=pl.BlockSpec((eblk // 2, 2 * H), lambda g: (g, 0)),
        out_shape=jax.ShapeDtypeStruct((EPAD // 2, 2 * H), jnp.bfloat16),
    )(ea_v, w2)

    edge_kernel, norm_kernel = _make_sc_kernels(H)
    ex, dn = edge_kernel(a2, b2, c2, i_pad, j_pad, ws_re)
    alpha = norm_kernel(dn, ex, i_pad)
    return alpha[:E]


# speedup vs baseline: 1.0236x; 1.0236x over previous
import functools

import jax
import jax.numpy as jnp
from jax import lax
from jax.experimental import pallas as pl
from jax.experimental.pallas import tpu as pltpu
from jax.experimental.pallas import tpu_sc as plsc

NPAD = 10240
TILES = 32
EPT = 10240
EPAD = TILES * EPT
K = 512
NB = EPT // K
NG = K // 16
NSB = NB // 2
NR = NPAD // 16

_CP = pltpu.CompilerParams(needs_layout_passes=False, use_tc_tiling_on_sc=False)


def _proj_nodes_kernel(x_ref, wi_ref, wj_ref, a_ref, b_ref):
    xv = x_ref[...]
    a_ref[...] = (2.0 * jnp.dot(xv, wi_ref[...],
                                preferred_element_type=jnp.float32)).astype(jnp.bfloat16)
    b_ref[...] = (2.0 * jnp.dot(xv, wj_ref[...],
                                preferred_element_type=jnp.float32)).astype(jnp.bfloat16)


def _proj_edges_kernel(ea_ref, we_ref, c_ref):
    c_ref[...] = (2.0 * jnp.dot(ea_ref[...], we_ref[...],
                                preferred_element_type=jnp.float32)).astype(jnp.bfloat16)


def _f32(x):
    return plsc.bitcast(x, jnp.float32)


def _make_sc_kernels(H):
    mesh = plsc.VectorSubcoreMesh(core_axis_name="c", subcore_axis_name="s")

    @functools.partial(
        pl.kernel,
        out_type=(
            jax.ShapeDtypeStruct((EPAD,), jnp.float32),
            jax.ShapeDtypeStruct((2, NR, 16), jnp.float32),
        ),
        mesh=mesh,
        scratch_types=(
            pltpu.VMEM((K,), jnp.int32),
            pltpu.VMEM((K,), jnp.int32),
            pltpu.VMEM((K,), jnp.int32),
            pltpu.VMEM((K,), jnp.int32),
            pltpu.VMEM((K,), jnp.int32),
            pltpu.VMEM((K, H), jnp.bfloat16),
            pltpu.VMEM((K, H), jnp.bfloat16),
            pltpu.VMEM((K, H), jnp.bfloat16),
            pltpu.VMEM((K, H), jnp.bfloat16),
            pltpu.VMEM((K, H), jnp.bfloat16),
            pltpu.VMEM((K, H), jnp.bfloat16),
            pltpu.VMEM((16 * K,), jnp.float32),
            pltpu.VMEM((K,), jnp.float32),
            pltpu.VMEM((K,), jnp.float32),
            pltpu.VMEM((NR, 16), jnp.float32),
            pltpu.VMEM((5, 128), jnp.int32),
            pltpu.VMEM((H,), jnp.float32),
            pltpu.VMEM_SHARED((NR, 16), jnp.float32),
            pltpu.SemaphoreType.DMA,
            pltpu.SemaphoreType.DMA,
            pltpu.SemaphoreType.DMA,
            pltpu.SemaphoreType.DMA,
            pltpu.SemaphoreType.DMA,
            pltpu.SemaphoreType.DMA,
        ),
        compiler_params=_CP,
    )
    def edge_kernel(a_hbm, b_hbm, c_hbm, i_hbm, j_hbm, ws_hbm,
                    ex_hbm, dn_hbm,
                    idxi0, idxi1, idxj0, idxj1, cidx,
                    bufa0, bufa1, bufb0, bufb1, bufc0, bufc1,
                    wbuf, exv0, exv1, dnv, rowidx, ws_v, shared_dn,
                    semi0, semi1, semg0, semg1, semo0, semo1):
        cid = lax.axis_index("c")
        sid = lax.axis_index("s")
        wid = cid * 16 + sid
        e0 = wid * EPT
        idxi = (idxi0, idxi1)
        idxj = (idxj0, idxj1)
        bufa = (bufa0, bufa1)
        bufb = (bufb0, bufb1)
        bufc = (bufc0, bufc1)
        exv = (exv0, exv1)
        semi = (semi0, semi1)
        semg = (semg0, semg1)
        semo = (semo0, semo1)

        def idx_descs(par, bb):
            eb = e0 + bb * K
            return (
                pltpu.make_async_copy(i_hbm.at[pl.ds(eb, K)], idxi[par], semi[par]),
                pltpu.make_async_copy(j_hbm.at[pl.ds(eb, K)], idxj[par], semi[par]),
            )

        def gather_descs(par, bb):
            eb = e0 + bb * K
            return (
                pltpu.make_async_copy(a_hbm.at[idxi[par]], bufa[par], semg[par]),
                pltpu.make_async_copy(b_hbm.at[idxj[par]], bufb[par], semg[par]),
                pltpu.make_async_copy(c_hbm.at[pl.ds(eb, K)], bufc[par], semg[par]),
            )

        def out_desc(par, bb):
            eb = e0 + bb * K
            return pltpu.make_async_copy(exv[par], ex_hbm.at[pl.ds(eb, K)], semo[par])

        pltpu.sync_copy(ws_hbm, ws_v)
        wsv = [ws_v[pl.ds(16 * k, 16)] for k in range(4)]
        ws2v = [2.0 * w for w in wsv]
        iota = lax.iota(jnp.int32, 16)
        iota_k = iota * K
        mask_hi = jnp.full((16,), -65536, jnp.int32)
        zero16 = jnp.zeros((16,), jnp.float32)

        @plsc.parallel_loop(0, NR)
        def _(r):
            dnv[r] = zero16

        for p in range(5):
            for o in range(8):
                rowidx[p, pl.ds(o * 16, 16)] = iota + (p * 128 + o * 16)

        @pl.when(sid == 0)
        def _():
            pltpu.sync_copy(dnv, shared_dn)
        plsc.subcore_barrier()

        def compute(par, bb):
            for o in range(NG):
                sl = pl.ds(o * 16, 16)
                cidx[sl] = idxi[par][sl]

            ba, bb_, bc = bufa[par], bufb[par], bufc[par]

            @plsc.parallel_loop(0, K, unroll=2)
            def _(e):
                acc = None
                for ch in range(2):
                    sl = pl.ds(32 * ch, 32)
                    va = plsc.bitcast(ba[e, sl], jnp.int32)
                    vb = plsc.bitcast(bb_[e, sl], jnp.int32)
                    vc = plsc.bitcast(bc[e, sl], jnp.int32)
                    for half in range(2):
                        if half == 0:
                            ae = _f32(va << 16)
                            be = _f32(vb << 16)
                            ce = _f32(vc << 16)
                        else:
                            ae = _f32(va & mask_hi)
                            be = _f32(vb & mask_hi)
                            ce = _f32(vc & mask_hi)
                        k = ch * 2 + half
                        s = (ae + be) + ce
                        t = jnp.exp(s)
                        r = wsv[k] - ws2v[k] / (t + 1.0)
                        acc = r if acc is None else acc + r
                plsc.store_scatter(wbuf, [iota_k + e], acc)

            ev = exv[par]

            @plsc.parallel_loop(0, NG, unroll=2)
            def _(g):
                gb = g * 16
                accs = [wbuf[pl.ds(l * K + gb, 16)] for l in range(4)]
                for q in range(1, 4):
                    for l in range(4):
                        accs[l] = accs[l] + wbuf[pl.ds((q * 4 + l) * K + gb, 16)]
                acc = (accs[0] + accs[1]) + (accs[2] + accs[3])
                ex16 = jnp.exp(acc)
                ev[pl.ds(gb, 16)] = ex16
                dst = cidx[pl.ds(gb, 16)]
                plsc.addupdate_scatter(dnv, [dst >> 4, dst & 15], ex16)

        for d in idx_descs(0, 0):
            d.start()
        for d in idx_descs(0, 0):
            d.wait()
        for d in gather_descs(0, 0):
            d.start()
        for d in idx_descs(1, 1):
            d.start()

        def hyper_body(hb, carry):
            for u in (0, 1):
                bb = hb * 2 + u
                par = u
                opar = 1 - u

                if u == 0:
                    for d in idx_descs(opar, bb + 1):
                        d.wait()
                    for d in gather_descs(opar, bb + 1):
                        d.start()
                else:
                    @pl.when(hb < NSB - 1)
                    def _():
                        for d in idx_descs(opar, bb + 1):
                            d.wait()
                        for d in gather_descs(opar, bb + 1):
                            d.start()

                for d in gather_descs(par, bb):
                    d.wait()

                @pl.when(hb >= 1)
                def _():
                    out_desc(par, bb - 2).wait()

                compute(par, bb)

                @pl.when(hb < NSB - 1)
                def _():
                    for d in idx_descs(par, bb + 2):
                        d.start()

                out_desc(par, bb).start()
            return carry

        lax.fori_loop(0, NSB, hyper_body, 0)
        out_desc(0, NB - 2).wait()
        out_desc(1, NB - 1).wait()

        for p in range(5):
            pltpu.sync_copy(dnv.at[pl.ds(p * 128, 128)],
                            shared_dn.at[rowidx.at[p]], add=True)
        plsc.subcore_barrier()
        pltpu.sync_copy(shared_dn.at[pl.ds(sid * (NR // 16), NR // 16)],
                        dn_hbm.at[cid, pl.ds(sid * (NR // 16), NR // 16)])

    @functools.partial(
        pl.kernel,
        out_type=jax.ShapeDtypeStruct((EPAD,), jnp.float32),
        mesh=mesh,
        scratch_types=(
            pltpu.VMEM((NR, 16), jnp.float32),
            pltpu.VMEM((NR, 16), jnp.float32),
            pltpu.VMEM((EPT,), jnp.float32),
            pltpu.VMEM((EPT,), jnp.int32),
            pltpu.VMEM((EPT,), jnp.float32),
            pltpu.SemaphoreType.DMA,
        ),
        compiler_params=_CP,
    )
    def norm_kernel(dn_hbm, ex_hbm, i_hbm, al_hbm,
                    dn0, dn1, ex_v, idx_v, al_v, sem):
        cid = lax.axis_index("c")
        sid = lax.axis_index("s")
        wid = cid * 16 + sid
        e0 = wid * EPT
        cp1 = pltpu.make_async_copy(dn_hbm.at[0], dn0, sem)
        cp2 = pltpu.make_async_copy(dn_hbm.at[1], dn1, sem)
        cp3 = pltpu.make_async_copy(ex_hbm.at[pl.ds(e0, EPT)], ex_v, sem)
        cp4 = pltpu.make_async_copy(i_hbm.at[pl.ds(e0, EPT)], idx_v, sem)
        for cp in (cp1, cp2, cp3, cp4):
            cp.start()
        cp1.wait()
        cp2.wait()

        @plsc.parallel_loop(0, NR)
        def _(r):
            dn0[r] = 1.0 / ((dn0[r] + dn1[r]) + 1e-16)

        cp3.wait()
        cp4.wait()

        @plsc.parallel_loop(0, EPT // 16, unroll=2)
        def _(g):
            sl = pl.ds(g * 16, 16)
            dst = idx_v[sl]
            inv = plsc.load_gather(dn0, [dst >> 4, dst & 15])
            al_v[sl] = ex_v[sl] * inv

        pltpu.sync_copy(al_v, al_hbm.at[pl.ds(e0, EPT)])

    return edge_kernel, norm_kernel


def kernel(x, edge_index, edge_attr, W_i, W_j, W_e, w_s):
    N, C = x.shape
    E, DE = edge_attr.shape
    H = W_i.shape[0]

    i_pad = jnp.concatenate(
        [edge_index[1], jnp.full((EPAD - E,), N, jnp.int32)], axis=0)
    j_pad = jnp.concatenate(
        [edge_index[0], jnp.zeros((EPAD - E,), jnp.int32)], axis=0)

    ws = w_s.reshape(H)
    ws_re = jnp.concatenate(
        [ws[0:32:2], ws[1:32:2], ws[32:64:2], ws[33:64:2]])

    nblk = 1024
    a2, b2 = pl.pallas_call(
        _proj_nodes_kernel,
        grid=(NPAD // nblk,),
        in_specs=[
            pl.BlockSpec((nblk, C), lambda g: (g, 0)),
            pl.BlockSpec((C, H), lambda g: (0, 0)),
            pl.BlockSpec((C, H), lambda g: (0, 0)),
        ],
        out_specs=[
            pl.BlockSpec((nblk, H), lambda g: (g, 0)),
            pl.BlockSpec((nblk, H), lambda g: (g, 0)),
        ],
        out_shape=[
            jax.ShapeDtypeStruct((NPAD, H), jnp.bfloat16),
            jax.ShapeDtypeStruct((NPAD, H), jnp.bfloat16),
        ],
    )(x, W_i.T, W_j.T)

    eblk = 16384
    c2 = pl.pallas_call(
        _proj_edges_kernel,
        grid=(EPAD // eblk,),
        in_specs=[
            pl.BlockSpec((eblk, DE), lambda g: (g, 0)),
            pl.BlockSpec((DE, H), lambda g: (0, 0)),
        ],
        out_specs=pl.BlockSpec((eblk, H), lambda g: (g, 0)),
        out_shape=jax.ShapeDtypeStruct((EPAD, H), jnp.bfloat16),
    )(edge_attr, W_e.T)

    edge_kernel, norm_kernel = _make_sc_kernels(H)
    ex, dn = edge_kernel(a2, b2, c2, i_pad, j_pad, ws_re)
    alpha = norm_kernel(dn, ex, i_pad)
    return alpha[:E]


# speedup vs baseline: 1.1474x; 1.1210x over previous
import functools

import jax
import jax.numpy as jnp
from jax import lax
from jax.experimental import pallas as pl
from jax.experimental.pallas import tpu as pltpu
from jax.experimental.pallas import tpu_sc as plsc

NPAD = 10240
TILES = 32
EPT = 10240
EPAD = TILES * EPT
K = 512
NB = EPT // K
NG = K // 16
NSB = NB // 2
NR = NPAD // 16

_CP = pltpu.CompilerParams(needs_layout_passes=False, use_tc_tiling_on_sc=False)


def _proj_nodes_kernel(x_ref, wi_ref, wj_ref, a_ref, b_ref):
    xv = x_ref[...]
    a_ref[...] = (2.0 * jnp.dot(xv, wi_ref[...],
                                preferred_element_type=jnp.float32)).astype(jnp.bfloat16)
    b_ref[...] = (2.0 * jnp.dot(xv, wj_ref[...],
                                preferred_element_type=jnp.float32)).astype(jnp.bfloat16)


def _proj_edges_kernel(ea_ref, w2_ref, c_ref):
    c_ref[...] = 2.0 * jnp.dot(ea_ref[...], w2_ref[...],
                               preferred_element_type=jnp.float32)


def _f32(x):
    return plsc.bitcast(x, jnp.float32)


def _make_sc_kernels(H):
    mesh = plsc.VectorSubcoreMesh(core_axis_name="c", subcore_axis_name="s")

    @functools.partial(
        pl.kernel,
        out_type=(
            jax.ShapeDtypeStruct((EPAD,), jnp.float32),
            jax.ShapeDtypeStruct((2, NR, 16), jnp.float32),
        ),
        mesh=mesh,
        scratch_types=(
            pltpu.VMEM((K,), jnp.int32),
            pltpu.VMEM((K,), jnp.int32),
            pltpu.VMEM((K,), jnp.int32),
            pltpu.VMEM((K,), jnp.int32),
            pltpu.VMEM((K,), jnp.int32),
            pltpu.VMEM((K, H), jnp.bfloat16),
            pltpu.VMEM((K, H), jnp.bfloat16),
            pltpu.VMEM((K, H), jnp.bfloat16),
            pltpu.VMEM((K, H), jnp.bfloat16),
            pltpu.VMEM((K // 2, 2 * H), jnp.float32),
            pltpu.VMEM((16 * K,), jnp.float32),
            pltpu.VMEM((K,), jnp.float32),
            pltpu.VMEM((K,), jnp.float32),
            pltpu.VMEM((NR, 16), jnp.float32),
            pltpu.VMEM((5, 128), jnp.int32),
            pltpu.VMEM((H,), jnp.float32),
            pltpu.VMEM_SHARED((NR, 16), jnp.float32),
            pltpu.SemaphoreType.DMA,
            pltpu.SemaphoreType.DMA,
            pltpu.SemaphoreType.DMA,
            pltpu.SemaphoreType.DMA,
            pltpu.SemaphoreType.DMA,
            pltpu.SemaphoreType.DMA,
            pltpu.SemaphoreType.DMA,
        ),
        compiler_params=_CP,
    )
    def edge_kernel(a_hbm, b_hbm, c_hbm, i_hbm, j_hbm, ws_hbm,
                    ex_hbm, dn_hbm,
                    idxi0, idxi1, idxj0, idxj1, cidx,
                    bufa0, bufa1, bufb0, bufb1, bufc,
                    wbuf, exv0, exv1, dnv, rowidx, ws_v, shared_dn,
                    semi0, semi1, semg0, semg1, semo0, semo1, semc):
        cid = lax.axis_index("c")
        sid = lax.axis_index("s")
        wid = cid * 16 + sid
        e0 = wid * EPT
        idxi = (idxi0, idxi1)
        idxj = (idxj0, idxj1)
        bufa = (bufa0, bufa1)
        bufb = (bufb0, bufb1)
        exv = (exv0, exv1)
        semi = (semi0, semi1)
        semg = (semg0, semg1)
        semo = (semo0, semo1)

        def idx_descs(par, bb):
            eb = e0 + bb * K
            return (
                pltpu.make_async_copy(i_hbm.at[pl.ds(eb, K)], idxi[par], semi[par]),
                pltpu.make_async_copy(j_hbm.at[pl.ds(eb, K)], idxj[par], semi[par]),
            )

        def gather_descs(par, bb):
            return (
                pltpu.make_async_copy(a_hbm.at[idxi[par]], bufa[par], semg[par]),
                pltpu.make_async_copy(b_hbm.at[idxj[par]], bufb[par], semg[par]),
            )

        def c_desc(bb):
            eb = e0 + bb * K
            return pltpu.make_async_copy(c_hbm.at[pl.ds(eb // 2, K // 2)],
                                         bufc, semc)

        def out_desc(par, bb):
            eb = e0 + bb * K
            return pltpu.make_async_copy(exv[par], ex_hbm.at[pl.ds(eb, K)], semo[par])

        pltpu.sync_copy(ws_hbm, ws_v)
        wsv = [ws_v[pl.ds(16 * k, 16)] for k in range(4)]
        ws2v = [2.0 * w for w in wsv]
        iota = lax.iota(jnp.int32, 16)
        iota_k = iota * K
        mask_hi = jnp.full((16,), -65536, jnp.int32)
        zero16 = jnp.zeros((16,), jnp.float32)

        @plsc.parallel_loop(0, NR)
        def _(r):
            dnv[r] = zero16

        for p in range(5):
            for o in range(8):
                rowidx[p, pl.ds(o * 16, 16)] = iota + (p * 128 + o * 16)

        @pl.when(sid == 0)
        def _():
            pltpu.sync_copy(dnv, shared_dn)
        plsc.subcore_barrier()

        def compute(par, bb):
            for o in range(NG):
                sl = pl.ds(o * 16, 16)
                cidx[sl] = idxi[par][sl]

            ba, bb_, bc = bufa[par], bufb[par], bufc

            @plsc.parallel_loop(0, K // 2, unroll=2)
            def _(rr):
                e2 = rr * 2
                for sub in range(2):
                    e = e2 + sub
                    acc = None
                    for ch in range(2):
                        sl = pl.ds(32 * ch, 32)
                        va = plsc.bitcast(ba[e, sl], jnp.int32)
                        vb = plsc.bitcast(bb_[e, sl], jnp.int32)
                        for half in range(2):
                            k = ch * 2 + half
                            ce = bc[rr, pl.ds(sub * 64 + 16 * k, 16)]
                            if half == 0:
                                ae = _f32(va << 16)
                                be = _f32(vb << 16)
                            else:
                                ae = _f32(va & mask_hi)
                                be = _f32(vb & mask_hi)
                            s = (ae + be) + ce
                            t = jnp.exp(s)
                            r = wsv[k] - ws2v[k] / (t + 1.0)
                            acc = r if acc is None else acc + r
                    plsc.store_scatter(wbuf, [iota_k + e], acc)

            ev = exv[par]

            @plsc.parallel_loop(0, NG, unroll=2)
            def _(g):
                gb = g * 16
                accs = [wbuf[pl.ds(l * K + gb, 16)] for l in range(4)]
                for q in range(1, 4):
                    for l in range(4):
                        accs[l] = accs[l] + wbuf[pl.ds((q * 4 + l) * K + gb, 16)]
                acc = (accs[0] + accs[1]) + (accs[2] + accs[3])
                ex16 = jnp.exp(acc)
                ev[pl.ds(gb, 16)] = ex16
                dst = cidx[pl.ds(gb, 16)]
                plsc.addupdate_scatter(dnv, [dst >> 4, dst & 15], ex16)

        for d in idx_descs(0, 0):
            d.start()
        for d in idx_descs(0, 0):
            d.wait()
        for d in gather_descs(0, 0):
            d.start()
        c_desc(0).start()
        for d in idx_descs(1, 1):
            d.start()

        def hyper_body(hb, carry):
            for u in (0, 1):
                bb = hb * 2 + u
                par = u
                opar = 1 - u

                if u == 0:
                    for d in idx_descs(opar, bb + 1):
                        d.wait()
                    for d in gather_descs(opar, bb + 1):
                        d.start()
                else:
                    @pl.when(hb < NSB - 1)
                    def _():
                        for d in idx_descs(opar, bb + 1):
                            d.wait()
                        for d in gather_descs(opar, bb + 1):
                            d.start()

                for d in gather_descs(par, bb):
                    d.wait()
                c_desc(bb).wait()

                @pl.when(hb >= 1)
                def _():
                    out_desc(par, bb - 2).wait()

                compute(par, bb)

                if u == 0:
                    c_desc(bb + 1).start()
                else:
                    @pl.when(hb < NSB - 1)
                    def _():
                        c_desc(bb + 1).start()

                @pl.when(hb < NSB - 1)
                def _():
                    for d in idx_descs(par, bb + 2):
                        d.start()

                out_desc(par, bb).start()
            return carry

        lax.fori_loop(0, NSB, hyper_body, 0)
        out_desc(0, NB - 2).wait()
        out_desc(1, NB - 1).wait()

        for p in range(5):
            pltpu.sync_copy(dnv.at[pl.ds(p * 128, 128)],
                            shared_dn.at[rowidx.at[p]], add=True)
        plsc.subcore_barrier()
        pltpu.sync_copy(shared_dn.at[pl.ds(sid * (NR // 16), NR // 16)],
                        dn_hbm.at[cid, pl.ds(sid * (NR // 16), NR // 16)])

    @functools.partial(
        pl.kernel,
        out_type=jax.ShapeDtypeStruct((EPAD,), jnp.float32),
        mesh=mesh,
        scratch_types=(
            pltpu.VMEM((NR, 16), jnp.float32),
            pltpu.VMEM((NR, 16), jnp.float32),
            pltpu.VMEM((EPT,), jnp.float32),
            pltpu.VMEM((EPT,), jnp.int32),
            pltpu.VMEM((EPT,), jnp.float32),
            pltpu.SemaphoreType.DMA,
        ),
        compiler_params=_CP,
    )
    def norm_kernel(dn_hbm, ex_hbm, i_hbm, al_hbm,
                    dn0, dn1, ex_v, idx_v, al_v, sem):
        cid = lax.axis_index("c")
        sid = lax.axis_index("s")
        wid = cid * 16 + sid
        e0 = wid * EPT
        cp1 = pltpu.make_async_copy(dn_hbm.at[0], dn0, sem)
        cp2 = pltpu.make_async_copy(dn_hbm.at[1], dn1, sem)
        cp3 = pltpu.make_async_copy(ex_hbm.at[pl.ds(e0, EPT)], ex_v, sem)
        cp4 = pltpu.make_async_copy(i_hbm.at[pl.ds(e0, EPT)], idx_v, sem)
        for cp in (cp1, cp2, cp3, cp4):
            cp.start()
        cp1.wait()
        cp2.wait()

        @plsc.parallel_loop(0, NR)
        def _(r):
            dn0[r] = 1.0 / ((dn0[r] + dn1[r]) + 1e-16)

        cp3.wait()
        cp4.wait()

        @plsc.parallel_loop(0, EPT // 16, unroll=2)
        def _(g):
            sl = pl.ds(g * 16, 16)
            dst = idx_v[sl]
            inv = plsc.load_gather(dn0, [dst >> 4, dst & 15])
            al_v[sl] = ex_v[sl] * inv

        pltpu.sync_copy(al_v, al_hbm.at[pl.ds(e0, EPT)])

    return edge_kernel, norm_kernel


def kernel(x, edge_index, edge_attr, W_i, W_j, W_e, w_s):
    N, C = x.shape
    E, DE = edge_attr.shape
    H = W_i.shape[0]

    i_pad = jnp.concatenate(
        [edge_index[1], jnp.full((EPAD - E,), N, jnp.int32)], axis=0)
    j_pad = jnp.concatenate(
        [edge_index[0], jnp.zeros((EPAD - E,), jnp.int32)], axis=0)

    ws = w_s.reshape(H)
    ws_re = jnp.concatenate(
        [ws[0:32:2], ws[1:32:2], ws[32:64:2], ws[33:64:2]])

    nblk = 1024
    a2, b2 = pl.pallas_call(
        _proj_nodes_kernel,
        grid=(NPAD // nblk,),
        in_specs=[
            pl.BlockSpec((nblk, C), lambda g: (g, 0)),
            pl.BlockSpec((C, H), lambda g: (0, 0)),
            pl.BlockSpec((C, H), lambda g: (0, 0)),
        ],
        out_specs=[
            pl.BlockSpec((nblk, H), lambda g: (g, 0)),
            pl.BlockSpec((nblk, H), lambda g: (g, 0)),
        ],
        out_shape=[
            jax.ShapeDtypeStruct((NPAD, H), jnp.bfloat16),
            jax.ShapeDtypeStruct((NPAD, H), jnp.bfloat16),
        ],
    )(x, W_i.T, W_j.T)

    wet = W_e.T
    zde = jnp.zeros((DE, H), jnp.float32)
    w2 = jnp.concatenate([
        jnp.concatenate([wet, zde], axis=1),
        jnp.concatenate([zde, wet], axis=1),
    ], axis=0)
    perm = jnp.concatenate([
        jnp.arange(0, 32, 2), jnp.arange(1, 32, 2),
        jnp.arange(32, 64, 2), jnp.arange(33, 64, 2)])
    perm2 = jnp.concatenate([perm, perm + H])
    w2 = w2[:, perm2]

    ea_v = edge_attr.reshape(E // 2, 2 * DE)
    eblk = 16384
    c2 = pl.pallas_call(
        _proj_edges_kernel,
        grid=(EPAD // eblk,),
        in_specs=[
            pl.BlockSpec((eblk // 2, 2 * DE), lambda g: (g, 0)),
            pl.BlockSpec((2 * DE, 2 * H), lambda g: (0, 0)),
        ],
        out_specs=pl.BlockSpec((eblk // 2, 2 * H), lambda g: (g, 0)),
        out_shape=jax.ShapeDtypeStruct((EPAD // 2, 2 * H), jnp.float32),
    )(ea_v, w2)

    edge_kernel, norm_kernel = _make_sc_kernels(H)
    ex, dn = edge_kernel(a2, b2, c2, i_pad, j_pad, ws_re)
    alpha = norm_kernel(dn, ex, i_pad)
    return alpha[:E]
